# baseline (speedup 1.0000x reference)
import jax
import jax.numpy as jnp
from jax.experimental import pallas as pl
from jax.experimental.pallas import tpu as pltpu

DIM = 128
BLOCK_ROWS = 32768


def _copy_body(x_ref, o_ref):
    o_ref[...] = x_ref[...]


def kernel(x):
    x = x.astype(jnp.float32)
    n = x.size // DIM
    x = x.reshape(n, DIM)
    grid = (n // BLOCK_ROWS,)
    return pl.pallas_call(
        _copy_body,
        out_shape=jax.ShapeDtypeStruct((n, DIM), jnp.float32),
        grid=grid,
        in_specs=[pl.BlockSpec((BLOCK_ROWS, DIM), lambda i: (i, 0))],
        out_specs=pl.BlockSpec((BLOCK_ROWS, DIM), lambda i: (i, 0)),
    )(x)


# speedup vs baseline: 1.1432x; 1.1432x over previous
import jax
import jax.numpy as jnp
from jax.experimental import pallas as pl
from jax.experimental.pallas import tpu as pltpu

DIM = 128
N_CHUNKS = 8


def _copy_body(x_hbm, o_hbm, vmem, in_sems, out_sems):
    rows = x_hbm.shape[0]
    chunk = rows // N_CHUNKS
    for i in range(N_CHUNKS):
        pltpu.make_async_copy(
            x_hbm.at[pl.ds(i * chunk, chunk)], vmem.at[i], in_sems.at[i]
        ).start()
    for i in range(N_CHUNKS):
        pltpu.make_async_copy(
            x_hbm.at[pl.ds(i * chunk, chunk)], vmem.at[i], in_sems.at[i]
        ).wait()
        pltpu.make_async_copy(
            vmem.at[i], o_hbm.at[pl.ds(i * chunk, chunk)], out_sems.at[i]
        ).start()
    for i in range(N_CHUNKS):
        pltpu.make_async_copy(
            vmem.at[i], o_hbm.at[pl.ds(i * chunk, chunk)], out_sems.at[i]
        ).wait()


def kernel(x):
    x = x.astype(jnp.float32)
    n = x.size // DIM
    x = x.reshape(n, DIM)
    chunk = n // N_CHUNKS
    return pl.pallas_call(
        _copy_body,
        out_shape=jax.ShapeDtypeStruct((n, DIM), jnp.float32),
        in_specs=[pl.BlockSpec(memory_space=pltpu.MemorySpace.HBM)],
        out_specs=pl.BlockSpec(memory_space=pltpu.MemorySpace.HBM),
        scratch_shapes=[
            pltpu.VMEM((N_CHUNKS, chunk, DIM), jnp.float32),
            pltpu.SemaphoreType.DMA((N_CHUNKS,)),
            pltpu.SemaphoreType.DMA((N_CHUNKS,)),
        ],
    )(x)
